# Initial kernel scaffold; baseline (speedup 1.0000x reference)
import jax
import jax.numpy as jnp
from jax.experimental import pallas as pl


def kernel(t, a, v, dia_len, W_fc, b_fc, W_hg, b_hg, type_emb):
    raise NotImplementedError("write your pallas kernel here")


# speedup vs baseline: 12.6910x; 12.6910x over previous
import jax
import jax.numpy as jnp
from jax.experimental import pallas as pl
from jax.experimental.pallas import tpu as pltpu

NUMD = 16
L = 120
N3 = 3 * L
D = 512
GD = 256
K = 30
TAUC = 0.5
U = NUMD * L
RB = 128
NRB = U // RB


def _stage_a_kernel(t_ref, a_ref, v_ref, wfc_ref, bfc_ref, whg_ref, bhg_ref,
                    te_ref, out_ref, pn_ref):
    f32 = jnp.float32
    F = jnp.concatenate([t_ref[:], a_ref[:], v_ref[:]], axis=0)
    sq = jnp.sum(F * F, axis=1, keepdims=True)
    G = jax.lax.dot_general(F, F, (((1,), (1,)), ((), ())),
                            precision=jax.lax.Precision.HIGHEST,
                            preferred_element_type=f32)
    d2 = sq + sq.reshape(1, N3) - 2.0 * G

    ci = jax.lax.broadcasted_iota(jnp.int32, (N3, N3), 1)
    ri = jax.lax.broadcasted_iota(jnp.int32, (N3, N3), 0)
    iota_f = ci.astype(f32)

    def body(_, carry):
        d2w, M = carry
        m = jnp.min(d2w, axis=1, keepdims=True)
        key = jnp.where(d2w <= m, iota_f, 1e9)
        am = jnp.min(key, axis=1, keepdims=True)
        one = (iota_f == am).astype(f32)
        return d2w + one * 1e30, M + one

    _, M = jax.lax.fori_loop(0, K, body, (d2, jnp.zeros((N3, N3), f32)))

    q = ri // 3
    anch = (ci == q) | (ci == q + L) | (ci == q + 2 * L)
    H = jnp.maximum(M, anch.astype(f32))

    X = jnp.dot(F, wfc_ref[:], preferred_element_type=f32) + bfc_ref[:]
    cnt_e = jnp.sum(H, axis=1, keepdims=True)
    te = te_ref[:]
    Te = jnp.concatenate([jnp.broadcast_to(te[0:1, :], (L, GD)),
                          jnp.broadcast_to(te[1:2, :], (L, GD)),
                          jnp.broadcast_to(te[2:3, :], (L, GD))], axis=0)
    Ef = (jnp.dot(H, X, preferred_element_type=f32)
          / jnp.maximum(cnt_e, 1.0)) + Te
    cnt_n = jax.lax.dot_general(H, jnp.ones((N3, 1), f32),
                                (((0,), (0,)), ((), ())),
                                preferred_element_type=f32)
    Nf = jax.lax.dot_general(H, Ef, (((0,), (0,)), ((), ())),
                             preferred_element_type=f32) / jnp.maximum(cnt_n, 1.0)
    R = jnp.maximum(jnp.dot(Nf, whg_ref[:], preferred_element_type=f32)
                    + bhg_ref[:], 0.0)
    out_ref[:, 0:GD] = R[0:L]
    out_ref[:, GD:2 * GD] = R[L:2 * L]
    out_ref[:, 2 * GD:3 * GD] = R[2 * L:3 * L]
    P = R / (jnp.sqrt(jnp.sum(R * R, axis=1, keepdims=True)) + 1e-8)
    pn_ref[:, 0:GD] = P[0:L]
    pn_ref[:, GD:2 * GD] = P[L:2 * L]
    pn_ref[:, 2 * GD:3 * GD] = P[2 * L:3 * L]


def _loss_kernel(x_ref, y_ref, out_ref, acc_ref, cm_ref, cs_ref):
    p = pl.program_id(0)
    r = pl.program_id(1)

    @pl.when(jnp.logical_and(p == 0, r == 0))
    def _():
        acc_ref[0] = 0.0

    @pl.when(r == 0)
    def _():
        acc_ref[1] = 0.0
        acc_ref[2] = 0.0
        cm_ref[:] = jnp.full((1, U), -1e30, jnp.float32)
        cs_ref[:] = jnp.zeros((1, U), jnp.float32)

    xb = x_ref[:]
    yn = y_ref[:]
    sim = jax.lax.dot_general(xb, yn, (((1,), (1,)), ((), ())),
                              preferred_element_type=jnp.float32) / TAUC
    rm = jnp.max(sim, axis=1, keepdims=True)
    row_lse = jnp.log(jnp.sum(jnp.exp(sim - rm), axis=1, keepdims=True)) + rm
    acc_ref[1] += jnp.sum(row_lse)
    yb = y_ref[pl.ds(r * RB, RB), :]
    diag = jnp.sum(xb * yb, axis=1) / TAUC
    acc_ref[2] += jnp.sum(diag)
    bm = jnp.max(sim, axis=0, keepdims=True)
    nm = jnp.maximum(cm_ref[:], bm)
    cs_ref[:] = (cs_ref[:] * jnp.exp(cm_ref[:] - nm)
                 + jnp.sum(jnp.exp(sim - nm), axis=0, keepdims=True))
    cm_ref[:] = nm

    @pl.when(r == NRB - 1)
    def _():
        col_lse = jnp.log(cs_ref[:]) + cm_ref[:]
        l1s = acc_ref[1] - acc_ref[2]
        l2s = jnp.sum(col_lse) - acc_ref[2]
        acc_ref[0] += 0.5 * (l1s + l2s) / U / 3.0

    @pl.when(jnp.logical_and(p == 2, r == NRB - 1))
    def _():
        out_ref[:, :] = jnp.full((1, 1), acc_ref[0], jnp.float32)


def kernel(t, a, v, dia_len, W_fc, b_fc, W_hg, b_hg, type_emb):
    del dia_len
    bfc = b_fc.reshape(1, GD).astype(jnp.float32)
    bhg = b_hg.reshape(1, GD).astype(jnp.float32)
    out, pn = pl.pallas_call(
        _stage_a_kernel,
        grid=(NUMD,),
        in_specs=[
            pl.BlockSpec((L, D), lambda d: (d, 0)),
            pl.BlockSpec((L, D), lambda d: (d, 0)),
            pl.BlockSpec((L, D), lambda d: (d, 0)),
            pl.BlockSpec((D, GD), lambda d: (0, 0)),
            pl.BlockSpec((1, GD), lambda d: (0, 0)),
            pl.BlockSpec((GD, GD), lambda d: (0, 0)),
            pl.BlockSpec((1, GD), lambda d: (0, 0)),
            pl.BlockSpec((3, GD), lambda d: (0, 0)),
        ],
        out_specs=[
            pl.BlockSpec((L, 3 * GD), lambda d: (d, 0)),
            pl.BlockSpec((L, 3 * GD), lambda d: (d, 0)),
        ],
        out_shape=[
            jax.ShapeDtypeStruct((U, 3 * GD), jnp.float32),
            jax.ShapeDtypeStruct((U, 3 * GD), jnp.float32),
        ],
    )(t, a, v, W_fc, bfc, W_hg, bhg, type_emb)

    loss = pl.pallas_call(
        _loss_kernel,
        grid=(3, NRB),
        in_specs=[
            pl.BlockSpec((RB, GD), lambda p, r: (r, p // 2)),
            pl.BlockSpec((U, GD), lambda p, r: (0, jnp.minimum(p + 1, 2))),
        ],
        out_specs=pl.BlockSpec((1, 1), lambda p, r: (0, 0)),
        out_shape=jax.ShapeDtypeStruct((1, 1), jnp.float32),
        scratch_shapes=[
            pltpu.SMEM((4,), jnp.float32),
            pltpu.VMEM((1, U), jnp.float32),
            pltpu.VMEM((1, U), jnp.float32),
        ],
    )(pn, pn)
    return out, loss.reshape(())


# speedup vs baseline: 33.8409x; 2.6665x over previous
import jax
import jax.numpy as jnp
from jax.experimental import pallas as pl
from jax.experimental.pallas import tpu as pltpu

NUMD = 16
L = 120
N3 = 3 * L
D = 512
GD = 256
K = 30
TAUC = 0.5
U = NUMD * L
RB = 384
NRB = U // RB


def _stage_a_kernel(t_ref, a_ref, v_ref, wfc_ref, bfc_ref, whg_ref, bhg_ref,
                    te_ref, out_ref, pn_ref):
    f32 = jnp.float32
    F = jnp.concatenate([t_ref[:], a_ref[:], v_ref[:]], axis=0)
    sq = jnp.sum(F * F, axis=1, keepdims=True)
    G = jax.lax.dot_general(F, F, (((1,), (1,)), ((), ())),
                            precision=jax.lax.Precision.HIGHEST,
                            preferred_element_type=f32)
    d2 = sq + sq.reshape(1, N3) - 2.0 * G

    ci = jax.lax.broadcasted_iota(jnp.int32, (N3, N3), 1)
    ri = jax.lax.broadcasted_iota(jnp.int32, (N3, N3), 0)

    m = jnp.min(d2, axis=1, keepdims=True)
    for _ in range(K - 1):
        m = jnp.min(jnp.where(d2 <= m, 1e30, d2), axis=1, keepdims=True)
    M = (d2 <= m).astype(f32)

    q = ri // 3
    anch = (ci == q) | (ci == q + L) | (ci == q + 2 * L)
    H = jnp.maximum(M, anch.astype(f32))

    X = jnp.dot(F, wfc_ref[:], preferred_element_type=f32) + bfc_ref[:]
    cnt_e = jnp.sum(H, axis=1, keepdims=True)
    te = te_ref[:]
    Te = jnp.concatenate([jnp.broadcast_to(te[0:1, :], (L, GD)),
                          jnp.broadcast_to(te[1:2, :], (L, GD)),
                          jnp.broadcast_to(te[2:3, :], (L, GD))], axis=0)
    Ef = (jnp.dot(H, X, preferred_element_type=f32)
          / jnp.maximum(cnt_e, 1.0)) + Te
    cnt_n = jax.lax.dot_general(H, jnp.ones((N3, 1), f32),
                                (((0,), (0,)), ((), ())),
                                preferred_element_type=f32)
    Nf = jax.lax.dot_general(H, Ef, (((0,), (0,)), ((), ())),
                             preferred_element_type=f32) / jnp.maximum(cnt_n, 1.0)
    R = jnp.maximum(jnp.dot(Nf, whg_ref[:], preferred_element_type=f32)
                    + bhg_ref[:], 0.0)
    out_ref[:, 0:GD] = R[0:L]
    out_ref[:, GD:2 * GD] = R[L:2 * L]
    out_ref[:, 2 * GD:3 * GD] = R[2 * L:3 * L]
    P = R / (jnp.sqrt(jnp.sum(R * R, axis=1, keepdims=True)) + 1e-8)
    pn_ref[:, 0:GD] = P[0:L]
    pn_ref[:, GD:2 * GD] = P[L:2 * L]
    pn_ref[:, 2 * GD:3 * GD] = P[2 * L:3 * L]


def _loss_kernel(x_ref, y_ref, out_ref, acc_ref, cs_ref):
    r = pl.program_id(1)

    @pl.when(r == 0)
    def _():
        acc_ref[0] = 0.0
        acc_ref[1] = 0.0
        cs_ref[:] = jnp.zeros((1, U), jnp.float32)

    xb = x_ref[:]
    yn = y_ref[:]
    sim = jax.lax.dot_general(xb, yn, (((1,), (1,)), ((), ())),
                              preferred_element_type=jnp.float32) * (1.0 / TAUC)
    P = jnp.exp(sim - 2.0)
    rs = jnp.dot(P, jnp.ones((U, 1), jnp.float32),
                 preferred_element_type=jnp.float32)
    acc_ref[0] += jnp.sum(jnp.log(rs))
    yb = y_ref[pl.ds(r * RB, RB), :]
    diag = jnp.sum(xb * yb, axis=1) * (1.0 / TAUC)
    acc_ref[1] += jnp.sum(diag)
    cs_ref[:] = cs_ref[:] + jnp.sum(P, axis=0, keepdims=True)

    @pl.when(r == NRB - 1)
    def _():
        row_sum = acc_ref[0] + 2.0 * U
        col_sum = jnp.sum(jnp.log(cs_ref[:])) + 2.0 * U
        pair = 0.5 * ((row_sum - acc_ref[1]) + (col_sum - acc_ref[1])) / U / 3.0
        out_ref[:, :, :] = jnp.full((1, 1, 1), pair, jnp.float32)


def kernel(t, a, v, dia_len, W_fc, b_fc, W_hg, b_hg, type_emb):
    del dia_len
    bfc = b_fc.reshape(1, GD).astype(jnp.float32)
    bhg = b_hg.reshape(1, GD).astype(jnp.float32)
    out, pn = pl.pallas_call(
        _stage_a_kernel,
        grid=(NUMD,),
        in_specs=[
            pl.BlockSpec((L, D), lambda d: (d, 0)),
            pl.BlockSpec((L, D), lambda d: (d, 0)),
            pl.BlockSpec((L, D), lambda d: (d, 0)),
            pl.BlockSpec((D, GD), lambda d: (0, 0)),
            pl.BlockSpec((1, GD), lambda d: (0, 0)),
            pl.BlockSpec((GD, GD), lambda d: (0, 0)),
            pl.BlockSpec((1, GD), lambda d: (0, 0)),
            pl.BlockSpec((3, GD), lambda d: (0, 0)),
        ],
        out_specs=[
            pl.BlockSpec((L, 3 * GD), lambda d: (d, 0)),
            pl.BlockSpec((L, 3 * GD), lambda d: (d, 0)),
        ],
        out_shape=[
            jax.ShapeDtypeStruct((U, 3 * GD), jnp.float32),
            jax.ShapeDtypeStruct((U, 3 * GD), jnp.float32),
        ],
        compiler_params=pltpu.CompilerParams(
            dimension_semantics=("parallel",)),
    )(t, a, v, W_fc, bfc, W_hg, bhg, type_emb)

    pair_losses = pl.pallas_call(
        _loss_kernel,
        grid=(3, NRB),
        in_specs=[
            pl.BlockSpec((RB, GD), lambda p, r: (r, p // 2)),
            pl.BlockSpec((U, GD), lambda p, r: (0, jnp.minimum(p + 1, 2))),
        ],
        out_specs=pl.BlockSpec((1, 1, 1), lambda p, r: (p, 0, 0)),
        out_shape=jax.ShapeDtypeStruct((3, 1, 1), jnp.float32),
        scratch_shapes=[
            pltpu.SMEM((2,), jnp.float32),
            pltpu.VMEM((1, U), jnp.float32),
        ],
        compiler_params=pltpu.CompilerParams(
            dimension_semantics=("parallel", "arbitrary")),
    )(pn, pn)
    return out, jnp.sum(pair_losses).reshape(())
